# baseline (speedup 1.0000x reference)
import functools

import jax
import jax.numpy as jnp
from jax import lax
from jax.experimental import pallas as pl
from jax.experimental.pallas import tpu as pltpu
from jax.experimental.pallas import tpu_sc as plsc

_NUM_CORES = 2
_NUM_SUBCORES = 16
_NUM_WORKERS = _NUM_CORES * _NUM_SUBCORES
_LANES = 16

_BATCH = 1024
_SEQ = 77
_HIDDEN = 768
_ROWS = _BATCH * _SEQ
_RPW = _ROWS // _NUM_WORKERS
_CHUNK = 56
_NCHUNKS = _RPW // _CHUNK


def _emb_body(ids_hbm, tok_hbm, pos_hbm, out_hbm, idx_v, pos_v, rows_v, sem):
    wid = lax.axis_index("s") * _NUM_CORES + lax.axis_index("c")
    row_base = wid * _RPW

    pltpu.sync_copy(ids_hbm.at[pl.ds(row_base, _RPW)], idx_v)
    pltpu.sync_copy(pos_hbm, pos_v)

    def chunk_step(j, carry):
        pltpu.async_copy(
            tok_hbm.at[idx_v.at[pl.ds(j * _CHUNK, _CHUNK)]], rows_v, sem
        ).wait()

        jm = lax.rem(j * _CHUNK, _SEQ)

        def row_add(i, c):
            p = jm + i
            p = p - jnp.where(p >= _SEQ, _SEQ, 0)
            for cstart in range(0, _HIDDEN, _LANES):
                sl = pl.ds(cstart, _LANES)
                rows_v[i, sl] = rows_v[i, sl] + pos_v[p, sl]
            return c

        lax.fori_loop(0, _CHUNK, row_add, 0)

        pltpu.sync_copy(rows_v, out_hbm.at[pl.ds(row_base + j * _CHUNK, _CHUNK)])
        return carry

    lax.fori_loop(0, _NCHUNKS, chunk_step, 0)


@jax.jit
def _emb_call(flat_ids, token_embedding, position_embedding):
    mesh = plsc.VectorSubcoreMesh(core_axis_name="c", subcore_axis_name="s")
    kern = functools.partial(
        pl.kernel,
        out_type=jax.ShapeDtypeStruct((_ROWS, _HIDDEN), jnp.float32),
        mesh=mesh,
        scratch_types=[
            pltpu.VMEM((_RPW,), jnp.int32),
            pltpu.VMEM((_SEQ, _HIDDEN), jnp.float32),
            pltpu.VMEM((_CHUNK, _HIDDEN), jnp.float32),
            pltpu.SemaphoreType.DMA,
        ],
    )(_emb_body)
    return kern(flat_ids, token_embedding, position_embedding)


def kernel(input_ids, token_embedding, position_embedding):
    input_shape = input_ids.shape
    seq_len = input_shape[-1]
    flat_ids = input_ids.reshape(-1).astype(jnp.int32)
    out = _emb_call(flat_ids, token_embedding, position_embedding)
    return out.reshape(-1, seq_len, token_embedding.shape[-1])


# speedup vs baseline: 1.8938x; 1.8938x over previous
import functools

import jax
import jax.numpy as jnp
from jax import lax
from jax.experimental import pallas as pl
from jax.experimental.pallas import tpu as pltpu
from jax.experimental.pallas import tpu_sc as plsc

_NUM_CORES = 2
_NUM_SUBCORES = 16
_NUM_WORKERS = _NUM_CORES * _NUM_SUBCORES
_LANES = 16

_BATCH = 1024
_SEQ = 77
_HIDDEN = 768
_NVEC = _HIDDEN // _LANES
_ROWS = _BATCH * _SEQ
_BPW = _BATCH // _NUM_WORKERS
_RPW = _ROWS // _NUM_WORKERS
_CHUNK = _BPW


def _scatter_groups(out_hbm, rows_v, half, wid, p, sem):
    descs = []
    for g in range(0, _CHUNK, _LANES):
        out_rows = (wid * _BPW + g + lax.iota(jnp.int32, _LANES)) * _SEQ + p
        descs.append(
            pltpu.make_async_copy(
                rows_v.at[pl.ds(half + g, _LANES)], out_hbm.at[out_rows], sem
            )
        )
    return descs


def _emb_body(ids_hbm, tok_hbm, pos_hbm, out_hbm, idx_v, pos_v, rows_v,
              sem_g0, sem_g1, sem_o0, sem_o1):
    wid = lax.axis_index("s") * _NUM_CORES + lax.axis_index("c")

    pltpu.sync_copy(ids_hbm.at[wid], idx_v)
    pltpu.sync_copy(pos_hbm, pos_v)

    def gather_desc(p, half, sem):
        return pltpu.make_async_copy(
            tok_hbm.at[idx_v.at[p]], rows_v.at[pl.ds(half, _CHUNK)], sem
        )

    gather_desc(0, 0, sem_g0).start()

    def outer(j2, carry):
        for b in (0, 1):
            j = 2 * j2 + b
            half = b * _CHUNK
            other = _CHUNK - half
            sem_g_cur = sem_g0 if b == 0 else sem_g1
            sem_g_nxt = sem_g1 if b == 0 else sem_g0
            sem_o_cur = sem_o0 if b == 0 else sem_o1
            sem_o_prv = sem_o1 if b == 0 else sem_o0

            @pl.when(j < _SEQ)
            def _chunk():
                @pl.when(j >= 1)
                def _():
                    for d in _scatter_groups(out_hbm, rows_v, other, wid,
                                             j - 1, sem_o_prv):
                        d.wait()

                @pl.when(j + 1 < _SEQ)
                def _():
                    gather_desc(j + 1, other, sem_g_nxt).start()

                gather_desc(j, half, sem_g_cur).wait()

                for cbase in range(0, _NVEC, _NVEC // 2):
                    pvecs = [pos_v[j, pl.ds((cbase + c) * _LANES, _LANES)]
                             for c in range(_NVEC // 2)]

                    def row_add(i, c, cbase=cbase, pvecs=pvecs):
                        r = half + i
                        for c2 in range(_NVEC // 2):
                            sl = pl.ds((cbase + c2) * _LANES, _LANES)
                            rows_v[r, sl] = rows_v[r, sl] + pvecs[c2]
                        return c

                    lax.fori_loop(0, _CHUNK, row_add, 0)

                for d in _scatter_groups(out_hbm, rows_v, half, wid, j,
                                         sem_o_cur):
                    d.start()

        return carry

    lax.fori_loop(0, (_SEQ + 1) // 2, outer, 0)

    for d in _scatter_groups(out_hbm, rows_v, 0, wid, _SEQ - 1, sem_o0):
        d.wait()


@jax.jit
def _emb_call(ids_pm, token_embedding, position_embedding):
    mesh = plsc.VectorSubcoreMesh(core_axis_name="c", subcore_axis_name="s")
    kern = functools.partial(
        pl.kernel,
        out_type=jax.ShapeDtypeStruct((_ROWS, _HIDDEN), jnp.float32),
        mesh=mesh,
        scratch_types=[
            pltpu.VMEM((_SEQ, _BPW), jnp.int32),
            pltpu.VMEM((_SEQ, _HIDDEN), jnp.float32),
            pltpu.VMEM((2 * _CHUNK, _HIDDEN), jnp.float32),
            pltpu.SemaphoreType.DMA,
            pltpu.SemaphoreType.DMA,
            pltpu.SemaphoreType.DMA,
            pltpu.SemaphoreType.DMA,
        ],
    )(_emb_body)
    return kern(ids_pm, token_embedding, position_embedding)


def kernel(input_ids, token_embedding, position_embedding):
    input_shape = input_ids.shape
    seq_len = input_shape[-1]
    flat_ids = input_ids.reshape(-1, seq_len).astype(jnp.int32)
    ids_pm = flat_ids.reshape(_NUM_WORKERS, _BPW, _SEQ).transpose(0, 2, 1)
    out = _emb_call(ids_pm, token_embedding, position_embedding)
    return out.reshape(-1, seq_len, token_embedding.shape[-1])


# speedup vs baseline: 4.5639x; 2.4099x over previous
import functools

import jax
import jax.numpy as jnp
from jax import lax
from jax.experimental import pallas as pl
from jax.experimental.pallas import tpu as pltpu
from jax.experimental.pallas import tpu_sc as plsc

_NUM_CORES = 2
_NUM_SUBCORES = 16
_NUM_WORKERS = _NUM_CORES * _NUM_SUBCORES
_LANES = 16

_BATCH = 1024
_SEQ = 77
_HIDDEN = 768
_NVEC = _HIDDEN // _LANES
_ROWS = _BATCH * _SEQ
_BPW = _BATCH // _NUM_WORKERS
_RPW = _ROWS // _NUM_WORKERS
_CHUNK = _BPW


def _out_desc(out_hbm, rows_v, half, wid, p, sem):
    return pltpu.make_async_copy(
        rows_v.at[pl.ds(half, _CHUNK)],
        out_hbm.at[pl.ds(p * _BATCH + wid * _BPW, _CHUNK)],
        sem,
    )


def _emb_body(ids_hbm, tok_hbm, pos_hbm, out_hbm, idx_v, pos_v, rows_v,
              sem_g0, sem_g1, sem_o0, sem_o1):
    wid = lax.axis_index("s") * _NUM_CORES + lax.axis_index("c")

    pltpu.sync_copy(ids_hbm.at[wid], idx_v)
    pltpu.sync_copy(pos_hbm, pos_v)

    def gather_desc(p, half, sem):
        return pltpu.make_async_copy(
            tok_hbm.at[idx_v.at[p]], rows_v.at[pl.ds(half, _CHUNK)], sem
        )

    gather_desc(0, 0, sem_g0).start()

    def outer(j2, carry):
        for b in (0, 1):
            j = 2 * j2 + b
            half = b * _CHUNK
            other = _CHUNK - half
            sem_g_cur = sem_g0 if b == 0 else sem_g1
            sem_g_nxt = sem_g1 if b == 0 else sem_g0
            sem_o_cur = sem_o0 if b == 0 else sem_o1
            sem_o_prv = sem_o1 if b == 0 else sem_o0

            @pl.when(j < _SEQ)
            def _chunk():
                @pl.when(j >= 1)
                def _():
                    _out_desc(out_hbm, rows_v, other, wid, j - 1,
                              sem_o_prv).wait()

                @pl.when(j + 1 < _SEQ)
                def _():
                    gather_desc(j + 1, other, sem_g_nxt).start()

                gather_desc(j, half, sem_g_cur).wait()

                for cbase in range(0, _NVEC, _NVEC // 2):
                    pvecs = [pos_v[j, pl.ds((cbase + c) * _LANES, _LANES)]
                             for c in range(_NVEC // 2)]

                    def row_add(i, c, cbase=cbase, pvecs=pvecs):
                        r = half + i
                        for c2 in range(_NVEC // 2):
                            sl = pl.ds((cbase + c2) * _LANES, _LANES)
                            rows_v[r, sl] = rows_v[r, sl] + pvecs[c2]
                        return c

                    lax.fori_loop(0, _CHUNK, row_add, 0)

                _out_desc(out_hbm, rows_v, half, wid, j, sem_o_cur).start()

        return carry

    lax.fori_loop(0, (_SEQ + 1) // 2, outer, 0)

    _out_desc(out_hbm, rows_v, 0, wid, _SEQ - 1, sem_o0).wait()


@jax.jit
def _emb_call(ids_pm, token_embedding, position_embedding):
    mesh = plsc.VectorSubcoreMesh(core_axis_name="c", subcore_axis_name="s")
    kern = functools.partial(
        pl.kernel,
        out_type=jax.ShapeDtypeStruct((_ROWS, _HIDDEN), jnp.float32),
        mesh=mesh,
        scratch_types=[
            pltpu.VMEM((_SEQ, _BPW), jnp.int32),
            pltpu.VMEM((_SEQ, _HIDDEN), jnp.float32),
            pltpu.VMEM((2 * _CHUNK, _HIDDEN), jnp.float32),
            pltpu.SemaphoreType.DMA,
            pltpu.SemaphoreType.DMA,
            pltpu.SemaphoreType.DMA,
            pltpu.SemaphoreType.DMA,
        ],
    )(_emb_body)
    return kern(ids_pm, token_embedding, position_embedding)


def kernel(input_ids, token_embedding, position_embedding):
    input_shape = input_ids.shape
    seq_len = input_shape[-1]
    flat_ids = input_ids.reshape(-1, seq_len).astype(jnp.int32)
    ids_pm = flat_ids.reshape(_NUM_WORKERS, _BPW, _SEQ).transpose(0, 2, 1)
    out = _emb_call(ids_pm, token_embedding, position_embedding)
    hidden = token_embedding.shape[-1]
    return out.reshape(seq_len, -1, hidden).transpose(1, 0, 2)


# speedup vs baseline: 5.5927x; 1.2254x over previous
import functools

import jax
import jax.numpy as jnp
from jax import lax
from jax.experimental import pallas as pl
from jax.experimental.pallas import tpu as pltpu
from jax.experimental.pallas import tpu_sc as plsc

_NUM_CORES = 2
_NUM_SUBCORES = 16
_NUM_WORKERS = _NUM_CORES * _NUM_SUBCORES
_LANES = 16

_BATCH = 1024
_SEQ = 77
_HIDDEN = 768
_NVEC = _HIDDEN // _LANES
_ROWS = _BATCH * _SEQ
_BPW = _BATCH // _NUM_WORKERS
_CHUNK = _BPW
_DEPTH = 4


def _emb_body(ids_hbm, tok_hbm, pos_hbm, out_hbm, idx_v, pos_v, rows_v,
              g0, g1, g2, g3, o0, o1, o2, o3):
    sem_g = [g0, g1, g2, g3]
    sem_o = [o0, o1, o2, o3]
    wid = lax.axis_index("s") * _NUM_CORES + lax.axis_index("c")

    pltpu.sync_copy(ids_hbm.at[wid], idx_v)

    def gather_desc(p, slot):
        return pltpu.make_async_copy(
            tok_hbm.at[idx_v.at[p]],
            rows_v.at[pl.ds(slot * _CHUNK, _CHUNK)],
            sem_g[slot],
        )

    def pos_desc(p, slot):
        return pltpu.make_async_copy(pos_hbm.at[p], pos_v.at[slot], sem_g[slot])

    def out_desc(p, slot):
        return pltpu.make_async_copy(
            rows_v.at[pl.ds(slot * _CHUNK, _CHUNK)],
            out_hbm.at[pl.ds(p * _BATCH + wid * _BPW, _CHUNK)],
            sem_o[slot],
        )

    for p in (0, 1):
        gather_desc(p, p).start()
        pos_desc(p, p).start()

    def outer(j4, carry):
        for b in range(_DEPTH):
            j = _DEPTH * j4 + b
            nxt = (b + 2) % _DEPTH

            @pl.when(j < _SEQ)
            def _chunk():
                gather_desc(j, b).wait()
                pos_desc(j, b).wait()

                for cbase in range(0, _NVEC, _NVEC // 2):
                    pvecs = [pos_v[b, pl.ds((cbase + c) * _LANES, _LANES)]
                             for c in range(_NVEC // 2)]

                    def row_add(i, c, pvecs=pvecs, cbase=cbase):
                        r = b * _CHUNK + i
                        for c2 in range(_NVEC // 2):
                            sl = pl.ds((cbase + c2) * _LANES, _LANES)
                            rows_v[r, sl] = rows_v[r, sl] + pvecs[c2]
                        return c

                    lax.fori_loop(0, _CHUNK, row_add, 0)

                out_desc(j, b).start()

                @pl.when(j >= 2)
                def _():
                    out_desc(j - 2, nxt).wait()

                @pl.when(j + 2 < _SEQ)
                def _():
                    gather_desc(j + 2, nxt).start()
                    pos_desc(j + 2, nxt).start()

        return carry

    lax.fori_loop(0, (_SEQ + _DEPTH - 1) // _DEPTH, outer, 0)

    out_desc(_SEQ - 2, (_SEQ - 2) % _DEPTH).wait()
    out_desc(_SEQ - 1, (_SEQ - 1) % _DEPTH).wait()


@jax.jit
def _emb_call(ids_pm, token_embedding, position_embedding):
    mesh = plsc.VectorSubcoreMesh(core_axis_name="c", subcore_axis_name="s")
    kern = functools.partial(
        pl.kernel,
        out_type=jax.ShapeDtypeStruct((_ROWS, _HIDDEN), jnp.float32),
        mesh=mesh,
        scratch_types=[
            pltpu.VMEM((_SEQ, _BPW), jnp.int32),
            pltpu.VMEM((_DEPTH, _HIDDEN), jnp.float32),
            pltpu.VMEM((_DEPTH * _CHUNK, _HIDDEN), jnp.float32),
            pltpu.SemaphoreType.DMA,
            pltpu.SemaphoreType.DMA,
            pltpu.SemaphoreType.DMA,
            pltpu.SemaphoreType.DMA,
            pltpu.SemaphoreType.DMA,
            pltpu.SemaphoreType.DMA,
            pltpu.SemaphoreType.DMA,
            pltpu.SemaphoreType.DMA,
        ],
    )(_emb_body)
    return kern(ids_pm, token_embedding, position_embedding)


def kernel(input_ids, token_embedding, position_embedding):
    input_shape = input_ids.shape
    seq_len = input_shape[-1]
    flat_ids = input_ids.reshape(-1, seq_len).astype(jnp.int32)
    ids_pm = flat_ids.reshape(_NUM_WORKERS, _BPW, _SEQ).transpose(0, 2, 1)
    out = _emb_call(ids_pm, token_embedding, position_embedding)
    hidden = token_embedding.shape[-1]
    return out.reshape(seq_len, -1, hidden).transpose(1, 0, 2)
